# baseline (speedup 1.0000x reference)
import jax
import jax.numpy as jnp
from jax import lax
from jax.experimental import pallas as pl
from jax.experimental.pallas import tpu as pltpu
from jax.experimental.pallas import tpu_sc as plsc

N = 100000
E = 100000
HID = 128
HEADS = 8
HD = 16
LAYERS = 4
OUTD = 64

NTILES = 32
C = 128
C2 = 512
NCH = (E + C - 1) // C
EU = NCH * C
EPAD = E + C
EPAD2 = E + C2 + C
SROWS = 16 * 6256
HPROWS = N + C

_PREC = jax.lax.Precision.HIGHEST
_NOTILE = pltpu.CompilerParams(use_tc_tiling_on_sc=False)


def _dot(a, b):
    return jax.lax.dot_general(a, b, (((1,), (0,)), ((), ())),
                               precision=_PREC, preferred_element_type=jnp.float32)


_RB = 1000
_NB = N // _RB


def _tc_stats_body(x_ref, w_ref, t_ref, st_ref, acc_ref):
    i = pl.program_id(0)
    t = _dot(x_ref[...], w_ref[...])
    t_ref[...] = t
    s0 = jnp.sum(t, axis=0, keepdims=True)
    s1 = jnp.sum(t * t, axis=0, keepdims=True)
    blk = jnp.concatenate([s0, s1], axis=0)

    @pl.when(i == 0)
    def _():
        acc_ref[...] = jnp.zeros_like(acc_ref)

    acc_ref[...] += blk
    st_ref[...] = acc_ref[...]


def _tc_stats(x, w):
    return pl.pallas_call(
        _tc_stats_body,
        grid=(_NB,),
        in_specs=[
            pl.BlockSpec((_RB, HID), lambda i: (i, 0)),
            pl.BlockSpec((HID, HID), lambda i: (0, 0)),
        ],
        out_specs=[
            pl.BlockSpec((_RB, HID), lambda i: (i, 0)),
            pl.BlockSpec((2, HID), lambda i: (0, 0)),
        ],
        out_shape=[
            jax.ShapeDtypeStruct((N, HID), jnp.float32),
            jax.ShapeDtypeStruct((2, HID), jnp.float32),
        ],
        scratch_shapes=[pltpu.VMEM((2, HID), jnp.float32)],
    )(x, w)


def _bn_act(t, st, g, b):
    mu = st[0:1, :] / N
    var = st[1:2, :] / N - mu * mu
    xn = (t - mu) * jax.lax.rsqrt(var + 1e-5)
    return jnp.maximum(g * xn + b, 0.0)


def _tc_embed_body(t_ref, st_ref, g_ref, b_ref, w1_ref, wg_ref, pr_ref,
                   x0_ref, hh_ref, elr_ref):
    a = _bn_act(t_ref[...], st_ref[...], g_ref[...], b_ref[...])
    x0 = _dot(a, w1_ref[...])
    x0_ref[...] = x0
    hh = _dot(x0, wg_ref[...])
    hh_ref[...] = hh
    elr_ref[...] = _dot(hh, pr_ref[...])


def _tc_embed(t, st, g, b, w1, wg, pr):
    return pl.pallas_call(
        _tc_embed_body,
        grid=(_NB,),
        in_specs=[
            pl.BlockSpec((_RB, HID), lambda i: (i, 0)),
            pl.BlockSpec((2, HID), lambda i: (0, 0)),
            pl.BlockSpec((1, HID), lambda i: (0, 0)),
            pl.BlockSpec((1, HID), lambda i: (0, 0)),
            pl.BlockSpec((HID, HID), lambda i: (0, 0)),
            pl.BlockSpec((HID, HID), lambda i: (0, 0)),
            pl.BlockSpec((HID, HID), lambda i: (0, 0)),
        ],
        out_specs=[
            pl.BlockSpec((_RB, HID), lambda i: (i, 0)),
            pl.BlockSpec((_RB, HID), lambda i: (i, 0)),
            pl.BlockSpec((_RB, HID), lambda i: (i, 0)),
        ],
        out_shape=[
            jax.ShapeDtypeStruct((N, HID), jnp.float32),
            jax.ShapeDtypeStruct((N, HID), jnp.float32),
            jax.ShapeDtypeStruct((N, HID), jnp.float32),
        ],
    )(t, st, g, b, w1, wg, pr)


def _tc_layer_body(hp_ref, x0_ref, wg_ref, pr_ref, hh_ref, elr_ref):
    hp = hp_ref[...]
    h = jnp.maximum(hp, 0.0) + 0.01 * jnp.minimum(hp, 0.0) + x0_ref[...]
    hh = _dot(h, wg_ref[...])
    hh_ref[...] = hh
    elr_ref[...] = _dot(hh, pr_ref[...])


def _tc_layer(hp, x0, wg, pr):
    return pl.pallas_call(
        _tc_layer_body,
        grid=(_NB,),
        in_specs=[
            pl.BlockSpec((_RB, HID), lambda i: (i, 0)),
            pl.BlockSpec((_RB, HID), lambda i: (i, 0)),
            pl.BlockSpec((HID, HID), lambda i: (0, 0)),
            pl.BlockSpec((HID, HID), lambda i: (0, 0)),
        ],
        out_specs=[
            pl.BlockSpec((_RB, HID), lambda i: (i, 0)),
            pl.BlockSpec((_RB, HID), lambda i: (i, 0)),
        ],
        out_shape=[
            jax.ShapeDtypeStruct((N, HID), jnp.float32),
            jax.ShapeDtypeStruct((N, HID), jnp.float32),
        ],
    )(hp, x0, wg, pr)


def _tc_dec_stats_body(hp_ref, x0_ref, w_ref, t_ref, st_ref, acc_ref):
    i = pl.program_id(0)
    hp = hp_ref[...]
    h = jnp.maximum(hp, 0.0) + 0.01 * jnp.minimum(hp, 0.0) + x0_ref[...]
    t = _dot(h, w_ref[...])
    t_ref[...] = t
    s0 = jnp.sum(t, axis=0, keepdims=True)
    s1 = jnp.sum(t * t, axis=0, keepdims=True)
    blk = jnp.concatenate([s0, s1], axis=0)

    @pl.when(i == 0)
    def _():
        acc_ref[...] = jnp.zeros_like(acc_ref)

    acc_ref[...] += blk
    st_ref[...] = acc_ref[...]


def _tc_dec_stats(hp, x0, w):
    return pl.pallas_call(
        _tc_dec_stats_body,
        grid=(_NB,),
        in_specs=[
            pl.BlockSpec((_RB, HID), lambda i: (i, 0)),
            pl.BlockSpec((_RB, HID), lambda i: (i, 0)),
            pl.BlockSpec((HID, HID), lambda i: (0, 0)),
        ],
        out_specs=[
            pl.BlockSpec((_RB, HID), lambda i: (i, 0)),
            pl.BlockSpec((2, HID), lambda i: (0, 0)),
        ],
        out_shape=[
            jax.ShapeDtypeStruct((N, HID), jnp.float32),
            jax.ShapeDtypeStruct((2, HID), jnp.float32),
        ],
        scratch_shapes=[pltpu.VMEM((2, HID), jnp.float32)],
    )(hp, x0, w)


def _tc_dec_out_body(t_ref, st_ref, g_ref, b_ref, w1_ref, o_ref):
    a = _bn_act(t_ref[...], st_ref[...], g_ref[...], b_ref[...])
    o_ref[...] = _dot(a, w1_ref[...])


def _tc_dec_out(t, st, g, b, w1):
    return pl.pallas_call(
        _tc_dec_out_body,
        grid=(_NB,),
        in_specs=[
            pl.BlockSpec((_RB, HID), lambda i: (i, 0)),
            pl.BlockSpec((2, HID), lambda i: (0, 0)),
            pl.BlockSpec((1, HID), lambda i: (0, 0)),
            pl.BlockSpec((1, HID), lambda i: (0, 0)),
            pl.BlockSpec((HID, OUTD), lambda i: (0, 0)),
        ],
        out_specs=pl.BlockSpec((_RB, OUTD), lambda i: (i, 0)),
        out_shape=jax.ShapeDtypeStruct((N, OUTD), jnp.float32),
    )(t, st, g, b, w1)


_MESH = plsc.VectorSubcoreMesh(core_axis_name="c", subcore_axis_name="s")


def _lane():
    return lax.iota(jnp.int32, 16)


def _sc_p1_body(a1_hbm, a2_hbm, elr_hbm, s_hbm, mx_hbm,
                i1_v, i2_v, r1_v, r2_v, s_v, m_v):
    w = lax.axis_index("s") * 2 + lax.axis_index("c")
    nck = (NCH + 31 - w) // 32
    neg = jnp.full((16,), -3.0e38, jnp.float32)
    head = _lane() < jnp.full((16,), HEADS, jnp.int32)

    def chunk(k, macc):
        base = (k * 32 + w) * C
        nv = jnp.minimum(C, E - base)
        pltpu.sync_copy(a1_hbm.at[pl.ds(base, C)], i1_v)
        pltpu.sync_copy(a2_hbm.at[pl.ds(base, C)], i2_v)
        pltpu.sync_copy(elr_hbm.at[i1_v], r1_v)
        pltpu.sync_copy(elr_hbm.at[i2_v], r2_v)

        def row(r, acc):
            x = r1_v[r, 0:16] + r2_v[r, 16:32]
            s = jnp.maximum(x, 0.0) + 0.2 * jnp.minimum(x, 0.0)
            s_v[r] = s
            return jnp.maximum(acc, jnp.where(head, s, neg))

        macc = lax.fori_loop(0, nv, row, macc)
        pltpu.sync_copy(s_v, s_hbm.at[pl.ds(base, C)])
        return macc

    macc = lax.fori_loop(0, nck, chunk, neg)
    m_v[0] = macc
    pltpu.sync_copy(m_v, mx_hbm.at[pl.ds(w, 1)])


def _sc_p1(a1, a2, elr):
    k = pl.kernel(
        _sc_p1_body,
        mesh=_MESH,
        out_type=[
            jax.ShapeDtypeStruct((EU, HD), jnp.float32),
            jax.ShapeDtypeStruct((NTILES, HD), jnp.float32),
        ],
        compiler_params=_NOTILE,
        scratch_types=[
            pltpu.VMEM((C,), jnp.int32),
            pltpu.VMEM((C,), jnp.int32),
            pltpu.VMEM((C, HID), jnp.float32),
            pltpu.VMEM((C, HID), jnp.float32),
            pltpu.VMEM((C, HD), jnp.float32),
            pltpu.VMEM((1, HD), jnp.float32),
        ],
    )
    return k(a1, a2, elr)


def _sc_p2_body(s_hbm, trg_hbm, ord_hbm, mx_hbm, e_hbm, sa_hbm, sb_hbm,
                sums_sh, s_v, e_v, trg_v, ord_v, mx_v, z_v):
    w = lax.axis_index("s") * 2 + lax.axis_index("c")
    core = lax.axis_index("c")
    sub = lax.axis_index("s")
    rows_per = SROWS // 16

    pltpu.sync_copy(mx_hbm, mx_v)
    macc = mx_v[0]
    for i in range(1, NTILES):
        macc = jnp.maximum(macc, mx_v[i])
    m = macc[0]
    for i in range(1, 16):
        m = jnp.maximum(m, macc[i])

    def zrow(r, _):
        z_v[r] = jnp.zeros((16,), jnp.float32)
        return 0

    lax.fori_loop(0, C, zrow, 0)
    nzc = rows_per // C
    rem = rows_per - nzc * C

    def zchunk(j, _):
        pltpu.sync_copy(z_v, sums_sh.at[pl.ds(sub * rows_per + j * C, C)])
        return 0

    lax.fori_loop(0, nzc, zchunk, 0)
    if rem:
        pltpu.sync_copy(z_v.at[pl.ds(0, rem)],
                        sums_sh.at[pl.ds(sub * rows_per + nzc * C, rem)])
    plsc.subcore_barrier()

    nck = (NCH + 31 - w) // 32

    def chunk(k, _):
        base = (k * 32 + w) * C
        pltpu.sync_copy(s_hbm.at[pl.ds(base, C)], s_v)
        pltpu.sync_copy(trg_hbm.at[pl.ds(base, C)], trg_v)
        pltpu.sync_copy(ord_hbm.at[pl.ds(base, C)], ord_v)

        def row(r, _):
            e_v[r] = jnp.exp(s_v[r] - m)
            return 0

        lax.fori_loop(0, C, row, 0)
        pltpu.sync_copy(e_v, sums_sh.at[trg_v], add=True)
        pltpu.sync_copy(e_v, e_hbm.at[ord_v])
        return 0

    lax.fori_loop(0, nck, chunk, 0)
    plsc.subcore_barrier()

    @pl.when(core == 0)
    def _():
        pltpu.sync_copy(sums_sh.at[pl.ds(sub * rows_per, rows_per)],
                        sa_hbm.at[pl.ds(sub * rows_per, rows_per)])

    @pl.when(core == 1)
    def _():
        pltpu.sync_copy(sums_sh.at[pl.ds(sub * rows_per, rows_per)],
                        sb_hbm.at[pl.ds(sub * rows_per, rows_per)])


def _sc_p2(s, trg_s, ord_s, mx):
    k = pl.kernel(
        _sc_p2_body,
        mesh=_MESH,
        out_type=[
            jax.ShapeDtypeStruct((EPAD, HD), jnp.float32),
            jax.ShapeDtypeStruct((SROWS, HD), jnp.float32),
            jax.ShapeDtypeStruct((SROWS, HD), jnp.float32),
        ],
        compiler_params=_NOTILE,
        scratch_types=[
            pltpu.VMEM_SHARED((SROWS, HD), jnp.float32),
            pltpu.VMEM((C, HD), jnp.float32),
            pltpu.VMEM((C, HD), jnp.float32),
            pltpu.VMEM((C,), jnp.int32),
            pltpu.VMEM((C,), jnp.int32),
            pltpu.VMEM((NTILES, HD), jnp.float32),
            pltpu.VMEM((C, HD), jnp.float32),
        ],
    )
    return k(s, trg_s, ord_s, mx)


def _bcast(v, h):
    idx = jnp.full((16,), h, jnp.int32).reshape(16, 1)
    dn = jax.lax.GatherDimensionNumbers(
        offset_dims=(), collapsed_slice_dims=(0,), start_index_map=(0,))
    return jax.lax.gather(v, idx, dn, (1,),
                          mode=jax.lax.GatherScatterMode.PROMISE_IN_BOUNDS)


def _sc_p3_body(tp_hbm, g1_hbm, g2_hbm, g3_hbm, seg_hbm, segn_hbm, keep_hbm,
                hh_hbm, e_hbm, sa_hbm, sb_hbm, hp_hbm,
                tp_v, g1_v, g2_v, g3_v, seg_v, segn_v, keep_v, ids_v,
                hh_v, e_v, sa_v, sb_v, stg_v, sem):
    w = lax.axis_index("s") * 2 + lax.axis_index("c")
    lane = _lane()
    nfull = jnp.full((16,), N, jnp.int32)

    pltpu.sync_copy(tp_hbm.at[pl.ds(w, 1)], tp_v)
    trow = tp_v[0]
    b0 = trow[0]
    b1 = trow[1]
    nb0 = trow[2]
    nb1 = trow[3]

    def zrow(r, _):
        for h in range(HEADS):
            stg_v[r, 16 * h:16 * (h + 1)] = jnp.zeros((16,), jnp.float32)
        return 0

    lax.fori_loop(0, C, zrow, 0)

    nz = nb1 - nb0
    nzc = nz // C

    def zchunk(j, _):
        pltpu.sync_copy(stg_v, hp_hbm.at[pl.ds(nb0 + j * C, C)])
        return 0

    lax.fori_loop(0, nzc, zchunk, 0)

    @pl.when((nz % C != 0) & (nz >= C))
    def _():
        pltpu.sync_copy(stg_v, hp_hbm.at[pl.ds(nb1 - C, C)])

    @pl.when(nz < C)
    def _():
        def zr(j, _):
            pltpu.sync_copy(stg_v.at[pl.ds(0, 1)], hp_hbm.at[pl.ds(nb0 + j, 1)])
            return 0

        lax.fori_loop(0, nz, zr, 0)

    ab0 = (b0 // 8) * 8
    nsup = (b1 - ab0 + C2 - 1) // C2

    def sup(sk, accs):
        sb_base = ab0 + sk * C2
        cps = [
            pltpu.async_copy(g1_hbm.at[pl.ds(sb_base, C2)], g1_v, sem),
            pltpu.async_copy(g2_hbm.at[pl.ds(sb_base, C2)], g2_v, sem),
            pltpu.async_copy(g3_hbm.at[pl.ds(sb_base, C2)], g3_v, sem),
            pltpu.async_copy(seg_hbm.at[pl.ds(sb_base, C2)], seg_v, sem),
            pltpu.async_copy(segn_hbm.at[pl.ds(sb_base, C2)], segn_v, sem),
            pltpu.async_copy(keep_hbm.at[pl.ds(sb_base, C2)], keep_v, sem),
        ]
        for cp in cps:
            cp.wait()
        gs = []
        for k in range(C2 // C):
            sl = pl.ds(k * C, C)
            gs.append(pltpu.async_copy(hh_hbm.at[g1_v.at[sl]], hh_v.at[sl], sem))
            gs.append(pltpu.async_copy(e_hbm.at[g2_v.at[sl]], e_v.at[sl], sem))
            gs.append(pltpu.async_copy(sa_hbm.at[g3_v.at[sl]], sa_v.at[sl], sem))
            gs.append(pltpu.async_copy(sb_hbm.at[g3_v.at[sl]], sb_v.at[sl], sem))
        for cp in gs:
            cp.wait()

        def sub(k, kaccs):
            cb = sb_base + k * C
            nv = jnp.clip(b1 - cb, 0, C)
            lo = jnp.clip(b0 - cb, 0, C)

            def edge(r, eaccs):
                rr = k * C + r
                kv = keep_v[rr]
                att = e_v[rr] / (sa_v[rr] + sb_v[rr] + 1e-16)
                new = []
                for h in range(HEADS):
                    hv = hh_v[rr, 16 * h:16 * (h + 1)]
                    ya = hv * _bcast(att, h) + kv * eaccs[h]
                    stg_v[r, 16 * h:16 * (h + 1)] = ya
                    new.append(ya)
                return tuple(new)

            kaccs = lax.fori_loop(lo, nv, edge, kaccs)

            for j in range(C // 16):
                rr = k * C + 16 * j
                sl16 = seg_v[pl.ds(rr, 16)]
                sn16 = segn_v[pl.ds(rr, 16)]
                pos = jnp.full((16,), 16 * j, jnp.int32) + lane
                valid = (pos >= jnp.full((16,), lo, jnp.int32)) & (
                    pos < jnp.full((16,), nv, jnp.int32))
                runend = (sl16 != sn16) & valid
                ids_v[16 * j:16 * j + 16] = jnp.where(runend, sl16, nfull)
            pltpu.sync_copy(stg_v, hp_hbm.at[ids_v])
            return kaccs

        return lax.fori_loop(0, C2 // C, sub, accs)

    init = tuple(jnp.zeros((16,), jnp.float32) for _ in range(HEADS))
    lax.fori_loop(0, nsup, sup, init)


def _sc_p3(tp, g1, g2, g3, seg, segn, keepv, hh, e, sa, sb):
    k = pl.kernel(
        _sc_p3_body,
        mesh=_MESH,
        out_type=jax.ShapeDtypeStruct((HPROWS, HID), jnp.float32),
        compiler_params=_NOTILE,
        scratch_types=[
            pltpu.VMEM((1, HD), jnp.int32),
            pltpu.VMEM((C2,), jnp.int32),
            pltpu.VMEM((C2,), jnp.int32),
            pltpu.VMEM((C2,), jnp.int32),
            pltpu.VMEM((C2,), jnp.int32),
            pltpu.VMEM((C2,), jnp.int32),
            pltpu.VMEM((C2, HD), jnp.float32),
            pltpu.VMEM((C,), jnp.int32),
            pltpu.VMEM((C2, HID), jnp.float32),
            pltpu.VMEM((C2, HD), jnp.float32),
            pltpu.VMEM((C2, HD), jnp.float32),
            pltpu.VMEM((C2, HD), jnp.float32),
            pltpu.VMEM((C, HID), jnp.float32),
            pltpu.SemaphoreType.DMA,
        ],
    )
    return k(tp, g1, g2, g3, seg, segn, keepv, hh, e, sa, sb)


def _pad_i32(x, length, fill):
    return jnp.concatenate(
        [x.astype(jnp.int32), jnp.full((length - x.shape[0],), fill, jnp.int32)])


def kernel(inputs, edge_index, embed_W0, embed_W1, embed_g, embed_b,
           gat_W, gat_al, gat_ar, dec_W0, dec_W1, dec_g, dec_b):
    src = edge_index[0].astype(jnp.int32)
    trg = edge_index[1].astype(jnp.int32)

    order = jnp.argsort(src).astype(jnp.int32)
    src_s = src[order]
    trg_p = trg[order]
    g1 = src[trg[order]]
    g3 = trg[trg[order]]
    segn = jnp.concatenate([src_s[1:], jnp.full((1,), N + 1, jnp.int32)])

    a1 = _pad_i32(src_s, EU, 0)
    a2g = _pad_i32(trg_p, EU, 0)
    a2s = _pad_i32(trg_p, EU, N)
    ord_pad = _pad_i32(order, EU, E)
    g1p = _pad_i32(g1, EPAD2, 0)
    g2p = _pad_i32(trg_p, EPAD2, 0)
    g3p = _pad_i32(g3, EPAD2, 0)
    segp = _pad_i32(src_s, EPAD2, N)
    segnp = _pad_i32(segn, EPAD2, N + 1)
    prev = jnp.concatenate([jnp.full((1,), -1, jnp.int32), src_s[:-1]])
    keep1 = (src_s == prev).astype(jnp.float32)
    keep16 = jnp.concatenate(
        [jnp.broadcast_to(keep1[:, None], (E, HD)),
         jnp.zeros((EPAD2 - E, HD), jnp.float32)])

    targ = (jnp.arange(1, NTILES, dtype=jnp.int32) * E) // NTILES
    vals = src_s[targ]
    bmid = jnp.searchsorted(src_s, vals, side="left").astype(jnp.int32)
    B = jnp.concatenate([jnp.zeros((1,), jnp.int32), bmid,
                         jnp.full((1,), E, jnp.int32)])
    node_b = jnp.where(B[:-1] < E, src_s[jnp.minimum(B[:-1], E - 1)], N)
    node_b = node_b.at[0].set(0)
    node_hi = jnp.concatenate([node_b[1:], jnp.full((1,), N, jnp.int32)])
    tp = jnp.zeros((NTILES, HD), jnp.int32)
    tp = tp.at[:, 0].set(B[:-1]).at[:, 1].set(B[1:])
    tp = tp.at[:, 2].set(node_b).at[:, 3].set(node_hi)

    def _proj(a):
        m = jnp.zeros((HID, HD), jnp.float32)
        hs = jnp.arange(HEADS)
        rows = (hs[:, None] * HD + jnp.arange(HD)[None, :]).reshape(-1)
        cols = jnp.repeat(hs, HD)
        return m.at[rows, cols].set(a.reshape(-1))

    def _prmat(al, ar):
        return jnp.concatenate(
            [_proj(al), _proj(ar), jnp.zeros((HID, HID - 2 * HD), jnp.float32)],
            axis=1)

    g1d = embed_g.reshape(1, HID)
    b1d = embed_b.reshape(1, HID)
    gd = dec_g.reshape(1, HID)
    bd = dec_b.reshape(1, HID)

    t, st = _tc_stats(inputs, embed_W0)
    x0, hh, elr = _tc_embed(t, st, g1d, b1d, embed_W1, gat_W[0],
                            _prmat(gat_al[0], gat_ar[0]))

    hp = None
    for l in range(LAYERS):
        s, mx = _sc_p1(a1, a2g, elr)
        e, sa, sb = _sc_p2(s, a2s, ord_pad, mx)
        hp = _sc_p3(tp, g1p, g2p, g3p, segp, segnp, keep16, hh, e, sa, sb)
        if l + 1 < LAYERS:
            hh, elr = _tc_layer(hp[:N], x0, gat_W[l + 1],
                                _prmat(gat_al[l + 1], gat_ar[l + 1]))

    t2, st2 = _tc_dec_stats(hp[:N], x0, dec_W0)
    out = _tc_dec_out(t2, st2, gd, bd, dec_W1)
    return out


# speedup vs baseline: 1.0259x; 1.0259x over previous
import jax
import jax.numpy as jnp
from jax import lax
from jax.experimental import pallas as pl
from jax.experimental.pallas import tpu as pltpu
from jax.experimental.pallas import tpu_sc as plsc

N = 100000
E = 100000
HID = 128
HEADS = 8
HD = 16
LAYERS = 4
OUTD = 64

NTILES = 32
C = 128
C2 = 256
NCH = (E + C - 1) // C
EU = NCH * C
EPAD = E + C
EPAD2 = E + C2 + C
SROWS = 16 * 6256
HPROWS = N + C

_PREC = jax.lax.Precision.HIGHEST
_NOTILE = pltpu.CompilerParams(use_tc_tiling_on_sc=False)


def _dot(a, b):
    return jax.lax.dot_general(a, b, (((1,), (0,)), ((), ())),
                               precision=_PREC, preferred_element_type=jnp.float32)


_RB = 1000
_NB = N // _RB


def _tc_stats_body(x_ref, w_ref, t_ref, st_ref, acc_ref):
    i = pl.program_id(0)
    t = _dot(x_ref[...], w_ref[...])
    t_ref[...] = t
    s0 = jnp.sum(t, axis=0, keepdims=True)
    s1 = jnp.sum(t * t, axis=0, keepdims=True)
    blk = jnp.concatenate([s0, s1], axis=0)

    @pl.when(i == 0)
    def _():
        acc_ref[...] = jnp.zeros_like(acc_ref)

    acc_ref[...] += blk
    st_ref[...] = acc_ref[...]


def _tc_stats(x, w):
    return pl.pallas_call(
        _tc_stats_body,
        grid=(_NB,),
        in_specs=[
            pl.BlockSpec((_RB, HID), lambda i: (i, 0)),
            pl.BlockSpec((HID, HID), lambda i: (0, 0)),
        ],
        out_specs=[
            pl.BlockSpec((_RB, HID), lambda i: (i, 0)),
            pl.BlockSpec((2, HID), lambda i: (0, 0)),
        ],
        out_shape=[
            jax.ShapeDtypeStruct((N, HID), jnp.float32),
            jax.ShapeDtypeStruct((2, HID), jnp.float32),
        ],
        scratch_shapes=[pltpu.VMEM((2, HID), jnp.float32)],
    )(x, w)


def _bn_act(t, st, g, b):
    mu = st[0:1, :] / N
    var = st[1:2, :] / N - mu * mu
    xn = (t - mu) * jax.lax.rsqrt(var + 1e-5)
    return jnp.maximum(g * xn + b, 0.0)


def _tc_embed_body(t_ref, st_ref, g_ref, b_ref, w1_ref, wg_ref, pr_ref,
                   x0_ref, hh_ref, elr_ref):
    a = _bn_act(t_ref[...], st_ref[...], g_ref[...], b_ref[...])
    x0 = _dot(a, w1_ref[...])
    x0_ref[...] = x0
    hh = _dot(x0, wg_ref[...])
    hh_ref[...] = hh
    elr_ref[...] = _dot(hh, pr_ref[...])


def _tc_embed(t, st, g, b, w1, wg, pr):
    return pl.pallas_call(
        _tc_embed_body,
        grid=(_NB,),
        in_specs=[
            pl.BlockSpec((_RB, HID), lambda i: (i, 0)),
            pl.BlockSpec((2, HID), lambda i: (0, 0)),
            pl.BlockSpec((1, HID), lambda i: (0, 0)),
            pl.BlockSpec((1, HID), lambda i: (0, 0)),
            pl.BlockSpec((HID, HID), lambda i: (0, 0)),
            pl.BlockSpec((HID, HID), lambda i: (0, 0)),
            pl.BlockSpec((HID, HID), lambda i: (0, 0)),
        ],
        out_specs=[
            pl.BlockSpec((_RB, HID), lambda i: (i, 0)),
            pl.BlockSpec((_RB, HID), lambda i: (i, 0)),
            pl.BlockSpec((_RB, HID), lambda i: (i, 0)),
        ],
        out_shape=[
            jax.ShapeDtypeStruct((N, HID), jnp.float32),
            jax.ShapeDtypeStruct((N, HID), jnp.float32),
            jax.ShapeDtypeStruct((N, HID), jnp.float32),
        ],
    )(t, st, g, b, w1, wg, pr)


def _tc_layer_body(hp_ref, x0_ref, wg_ref, pr_ref, hh_ref, elr_ref):
    hp = hp_ref[...]
    h = jnp.maximum(hp, 0.0) + 0.01 * jnp.minimum(hp, 0.0) + x0_ref[...]
    hh = _dot(h, wg_ref[...])
    hh_ref[...] = hh
    elr_ref[...] = _dot(hh, pr_ref[...])


def _tc_layer(hp, x0, wg, pr):
    return pl.pallas_call(
        _tc_layer_body,
        grid=(_NB,),
        in_specs=[
            pl.BlockSpec((_RB, HID), lambda i: (i, 0)),
            pl.BlockSpec((_RB, HID), lambda i: (i, 0)),
            pl.BlockSpec((HID, HID), lambda i: (0, 0)),
            pl.BlockSpec((HID, HID), lambda i: (0, 0)),
        ],
        out_specs=[
            pl.BlockSpec((_RB, HID), lambda i: (i, 0)),
            pl.BlockSpec((_RB, HID), lambda i: (i, 0)),
        ],
        out_shape=[
            jax.ShapeDtypeStruct((N, HID), jnp.float32),
            jax.ShapeDtypeStruct((N, HID), jnp.float32),
        ],
    )(hp, x0, wg, pr)


def _tc_dec_stats_body(hp_ref, x0_ref, w_ref, t_ref, st_ref, acc_ref):
    i = pl.program_id(0)
    hp = hp_ref[...]
    h = jnp.maximum(hp, 0.0) + 0.01 * jnp.minimum(hp, 0.0) + x0_ref[...]
    t = _dot(h, w_ref[...])
    t_ref[...] = t
    s0 = jnp.sum(t, axis=0, keepdims=True)
    s1 = jnp.sum(t * t, axis=0, keepdims=True)
    blk = jnp.concatenate([s0, s1], axis=0)

    @pl.when(i == 0)
    def _():
        acc_ref[...] = jnp.zeros_like(acc_ref)

    acc_ref[...] += blk
    st_ref[...] = acc_ref[...]


def _tc_dec_stats(hp, x0, w):
    return pl.pallas_call(
        _tc_dec_stats_body,
        grid=(_NB,),
        in_specs=[
            pl.BlockSpec((_RB, HID), lambda i: (i, 0)),
            pl.BlockSpec((_RB, HID), lambda i: (i, 0)),
            pl.BlockSpec((HID, HID), lambda i: (0, 0)),
        ],
        out_specs=[
            pl.BlockSpec((_RB, HID), lambda i: (i, 0)),
            pl.BlockSpec((2, HID), lambda i: (0, 0)),
        ],
        out_shape=[
            jax.ShapeDtypeStruct((N, HID), jnp.float32),
            jax.ShapeDtypeStruct((2, HID), jnp.float32),
        ],
        scratch_shapes=[pltpu.VMEM((2, HID), jnp.float32)],
    )(hp, x0, w)


def _tc_dec_out_body(t_ref, st_ref, g_ref, b_ref, w1_ref, o_ref):
    a = _bn_act(t_ref[...], st_ref[...], g_ref[...], b_ref[...])
    o_ref[...] = _dot(a, w1_ref[...])


def _tc_dec_out(t, st, g, b, w1):
    return pl.pallas_call(
        _tc_dec_out_body,
        grid=(_NB,),
        in_specs=[
            pl.BlockSpec((_RB, HID), lambda i: (i, 0)),
            pl.BlockSpec((2, HID), lambda i: (0, 0)),
            pl.BlockSpec((1, HID), lambda i: (0, 0)),
            pl.BlockSpec((1, HID), lambda i: (0, 0)),
            pl.BlockSpec((HID, OUTD), lambda i: (0, 0)),
        ],
        out_specs=pl.BlockSpec((_RB, OUTD), lambda i: (i, 0)),
        out_shape=jax.ShapeDtypeStruct((N, OUTD), jnp.float32),
    )(t, st, g, b, w1)


_MESH = plsc.VectorSubcoreMesh(core_axis_name="c", subcore_axis_name="s")


def _lane():
    return lax.iota(jnp.int32, 16)


def _sc_p1_body(a1_hbm, a2_hbm, elr_hbm, s_hbm, mx_hbm,
                i1_v, i2_v, r1_v, r2_v, s_v, m_v):
    w = lax.axis_index("s") * 2 + lax.axis_index("c")
    nck = (NCH + 31 - w) // 32
    neg = jnp.full((16,), -3.0e38, jnp.float32)
    head = _lane() < jnp.full((16,), HEADS, jnp.int32)

    def chunk(k, macc):
        base = (k * 32 + w) * C
        nv = jnp.minimum(C, E - base)
        pltpu.sync_copy(a1_hbm.at[pl.ds(base, C)], i1_v)
        pltpu.sync_copy(a2_hbm.at[pl.ds(base, C)], i2_v)
        pltpu.sync_copy(elr_hbm.at[i1_v], r1_v)
        pltpu.sync_copy(elr_hbm.at[i2_v], r2_v)

        def row(r, acc):
            x = r1_v[r, 0:16] + r2_v[r, 16:32]
            s = jnp.maximum(x, 0.0) + 0.2 * jnp.minimum(x, 0.0)
            s_v[r] = s
            return jnp.maximum(acc, jnp.where(head, s, neg))

        macc = lax.fori_loop(0, nv, row, macc)
        pltpu.sync_copy(s_v, s_hbm.at[pl.ds(base, C)])
        return macc

    macc = lax.fori_loop(0, nck, chunk, neg)
    m_v[0] = macc
    pltpu.sync_copy(m_v, mx_hbm.at[pl.ds(w, 1)])


def _sc_p1(a1, a2, elr):
    k = pl.kernel(
        _sc_p1_body,
        mesh=_MESH,
        out_type=[
            jax.ShapeDtypeStruct((EU, HD), jnp.float32),
            jax.ShapeDtypeStruct((NTILES, HD), jnp.float32),
        ],
        compiler_params=_NOTILE,
        scratch_types=[
            pltpu.VMEM((C,), jnp.int32),
            pltpu.VMEM((C,), jnp.int32),
            pltpu.VMEM((C, HID), jnp.float32),
            pltpu.VMEM((C, HID), jnp.float32),
            pltpu.VMEM((C, HD), jnp.float32),
            pltpu.VMEM((1, HD), jnp.float32),
        ],
    )
    return k(a1, a2, elr)


def _sc_p2_body(s_hbm, trg_hbm, ord_hbm, mx_hbm, e_hbm, sa_hbm, sb_hbm,
                sums_sh, s_v, e_v, trg_v, ord_v, mx_v, z_v):
    w = lax.axis_index("s") * 2 + lax.axis_index("c")
    core = lax.axis_index("c")
    sub = lax.axis_index("s")
    rows_per = SROWS // 16

    pltpu.sync_copy(mx_hbm, mx_v)
    macc = mx_v[0]
    for i in range(1, NTILES):
        macc = jnp.maximum(macc, mx_v[i])
    m = macc[0]
    for i in range(1, 16):
        m = jnp.maximum(m, macc[i])

    def zrow(r, _):
        z_v[r] = jnp.zeros((16,), jnp.float32)
        return 0

    lax.fori_loop(0, C, zrow, 0)
    nzc = rows_per // C
    rem = rows_per - nzc * C

    def zchunk(j, _):
        pltpu.sync_copy(z_v, sums_sh.at[pl.ds(sub * rows_per + j * C, C)])
        return 0

    lax.fori_loop(0, nzc, zchunk, 0)
    if rem:
        pltpu.sync_copy(z_v.at[pl.ds(0, rem)],
                        sums_sh.at[pl.ds(sub * rows_per + nzc * C, rem)])
    plsc.subcore_barrier()

    nck = (NCH + 31 - w) // 32

    def chunk(k, _):
        base = (k * 32 + w) * C
        pltpu.sync_copy(s_hbm.at[pl.ds(base, C)], s_v)
        pltpu.sync_copy(trg_hbm.at[pl.ds(base, C)], trg_v)
        pltpu.sync_copy(ord_hbm.at[pl.ds(base, C)], ord_v)

        def row(r, _):
            e_v[r] = jnp.exp(s_v[r] - m)
            return 0

        lax.fori_loop(0, C, row, 0)
        pltpu.sync_copy(e_v, sums_sh.at[trg_v], add=True)
        pltpu.sync_copy(e_v, e_hbm.at[ord_v])
        return 0

    lax.fori_loop(0, nck, chunk, 0)
    plsc.subcore_barrier()

    @pl.when(core == 0)
    def _():
        pltpu.sync_copy(sums_sh.at[pl.ds(sub * rows_per, rows_per)],
                        sa_hbm.at[pl.ds(sub * rows_per, rows_per)])

    @pl.when(core == 1)
    def _():
        pltpu.sync_copy(sums_sh.at[pl.ds(sub * rows_per, rows_per)],
                        sb_hbm.at[pl.ds(sub * rows_per, rows_per)])


def _sc_p2(s, trg_s, ord_s, mx):
    k = pl.kernel(
        _sc_p2_body,
        mesh=_MESH,
        out_type=[
            jax.ShapeDtypeStruct((EPAD, HD), jnp.float32),
            jax.ShapeDtypeStruct((SROWS, HD), jnp.float32),
            jax.ShapeDtypeStruct((SROWS, HD), jnp.float32),
        ],
        compiler_params=_NOTILE,
        scratch_types=[
            pltpu.VMEM_SHARED((SROWS, HD), jnp.float32),
            pltpu.VMEM((C, HD), jnp.float32),
            pltpu.VMEM((C, HD), jnp.float32),
            pltpu.VMEM((C,), jnp.int32),
            pltpu.VMEM((C,), jnp.int32),
            pltpu.VMEM((NTILES, HD), jnp.float32),
            pltpu.VMEM((C, HD), jnp.float32),
        ],
    )
    return k(s, trg_s, ord_s, mx)


def _sc_p2b_body(e_hbm, trg_hbm, sa_hbm, sb_hbm, att_hbm,
                 e_v, trg_v, sa_v, sb_v, att_v, sem):
    w = lax.axis_index("s") * 2 + lax.axis_index("c")
    nck = (NCH + 31 - w) // 32

    def chunk(k, _):
        base = (k * 32 + w) * C
        cps = [
            pltpu.async_copy(e_hbm.at[pl.ds(base, C)], e_v, sem),
            pltpu.async_copy(trg_hbm.at[pl.ds(base, C)], trg_v, sem),
        ]
        for cp in cps:
            cp.wait()
        gs = [
            pltpu.async_copy(sa_hbm.at[trg_v], sa_v, sem),
            pltpu.async_copy(sb_hbm.at[trg_v], sb_v, sem),
        ]
        for cp in gs:
            cp.wait()

        def row(r, _):
            att_v[r] = e_v[r] / (sa_v[r] + sb_v[r] + 1e-16)
            return 0

        lax.fori_loop(0, C, row, 0)
        pltpu.sync_copy(att_v, att_hbm.at[pl.ds(base, C)])
        return 0

    lax.fori_loop(0, nck, chunk, 0)


def _sc_p2b(e, trg_o, sa, sb):
    k = pl.kernel(
        _sc_p2b_body,
        mesh=_MESH,
        out_type=jax.ShapeDtypeStruct((EU, HD), jnp.float32),
        compiler_params=_NOTILE,
        scratch_types=[
            pltpu.VMEM((C, HD), jnp.float32),
            pltpu.VMEM((C,), jnp.int32),
            pltpu.VMEM((C, HD), jnp.float32),
            pltpu.VMEM((C, HD), jnp.float32),
            pltpu.VMEM((C, HD), jnp.float32),
            pltpu.SemaphoreType.DMA,
        ],
    )
    return k(e, trg_o, sa, sb)


def _tc_att_expand_body(a_ref, r_ref, o_ref):
    o_ref[...] = _dot(a_ref[...], r_ref[...])


def _tc_att_expand(att16, rmat):
    return pl.pallas_call(
        _tc_att_expand_body,
        grid=(_NB,),
        in_specs=[
            pl.BlockSpec((_RB, HD), lambda i: (i, 0)),
            pl.BlockSpec((HD, HID), lambda i: (0, 0)),
        ],
        out_specs=pl.BlockSpec((_RB, HID), lambda i: (i, 0)),
        out_shape=jax.ShapeDtypeStruct((E, HID), jnp.float32),
    )(att16, rmat)


def _sc_p3_body(tp_hbm, g1_hbm, g2_hbm, seg_hbm, segn_hbm, keep_hbm,
                hh_hbm, att_hbm, hp_hbm,
                tp_v, g1_v, g2_v, seg_v, segn_v, keep_v, ids_v,
                hh_v, att_v, stg_v, sem):
    w = lax.axis_index("s") * 2 + lax.axis_index("c")
    lane = _lane()
    nfull = jnp.full((16,), N, jnp.int32)

    pltpu.sync_copy(tp_hbm.at[pl.ds(w, 1)], tp_v)
    trow = tp_v[0]
    b0 = trow[0]
    b1 = trow[1]
    nb0 = trow[2]
    nb1 = trow[3]

    def zrow(r, _):
        for h in range(HEADS):
            stg_v[r, 16 * h:16 * (h + 1)] = jnp.zeros((16,), jnp.float32)
        return 0

    lax.fori_loop(0, C, zrow, 0)

    nz = nb1 - nb0
    nzc = nz // C

    def zchunk(j, _):
        pltpu.sync_copy(stg_v, hp_hbm.at[pl.ds(nb0 + j * C, C)])
        return 0

    lax.fori_loop(0, nzc, zchunk, 0)

    @pl.when((nz % C != 0) & (nz >= C))
    def _():
        pltpu.sync_copy(stg_v, hp_hbm.at[pl.ds(nb1 - C, C)])

    @pl.when(nz < C)
    def _():
        def zr(j, _):
            pltpu.sync_copy(stg_v.at[pl.ds(0, 1)], hp_hbm.at[pl.ds(nb0 + j, 1)])
            return 0

        lax.fori_loop(0, nz, zr, 0)

    ab0 = (b0 // 8) * 8
    nsup = (b1 - ab0 + C2 - 1) // C2

    def sup(sk, accs):
        sb_base = ab0 + sk * C2
        cps = [
            pltpu.async_copy(g1_hbm.at[pl.ds(sb_base, C2)], g1_v, sem),
            pltpu.async_copy(g2_hbm.at[pl.ds(sb_base, C2)], g2_v, sem),
            pltpu.async_copy(seg_hbm.at[pl.ds(sb_base, C2)], seg_v, sem),
            pltpu.async_copy(segn_hbm.at[pl.ds(sb_base, C2)], segn_v, sem),
            pltpu.async_copy(keep_hbm.at[pl.ds(sb_base, C2)], keep_v, sem),
        ]
        for cp in cps:
            cp.wait()
        gs = []
        for k in range(C2 // C):
            sl = pl.ds(k * C, C)
            gs.append(pltpu.async_copy(hh_hbm.at[g1_v.at[sl]], hh_v.at[sl], sem))
            gs.append(pltpu.async_copy(att_hbm.at[g2_v.at[sl]], att_v.at[sl], sem))
        for cp in gs:
            cp.wait()

        def sub(k, kaccs):
            cb = sb_base + k * C
            nv = jnp.clip(b1 - cb, 0, C)
            lo = jnp.clip(b0 - cb, 0, C)

            def edge(r, eaccs):
                rr = k * C + r
                kv = keep_v[rr]
                new = []
                for h in range(HEADS):
                    hv = hh_v[rr, 16 * h:16 * (h + 1)]
                    av = att_v[rr, 16 * h:16 * (h + 1)]
                    ya = hv * av + kv * eaccs[h]
                    stg_v[r, 16 * h:16 * (h + 1)] = ya
                    new.append(ya)
                return tuple(new)

            kaccs = lax.fori_loop(lo, nv, edge, kaccs)

            for j in range(C // 16):
                rr = k * C + 16 * j
                sl16 = seg_v[pl.ds(rr, 16)]
                sn16 = segn_v[pl.ds(rr, 16)]
                pos = jnp.full((16,), 16 * j, jnp.int32) + lane
                valid = (pos >= jnp.full((16,), lo, jnp.int32)) & (
                    pos < jnp.full((16,), nv, jnp.int32))
                runend = (sl16 != sn16) & valid
                ids_v[16 * j:16 * j + 16] = jnp.where(runend, sl16, nfull)
            pltpu.sync_copy(stg_v, hp_hbm.at[ids_v])
            return kaccs

        return lax.fori_loop(0, C2 // C, sub, accs)

    init = tuple(jnp.zeros((16,), jnp.float32) for _ in range(HEADS))
    lax.fori_loop(0, nsup, sup, init)


def _sc_p3(tp, g1, g2, seg, segn, keepv, hh, att):
    k = pl.kernel(
        _sc_p3_body,
        mesh=_MESH,
        out_type=jax.ShapeDtypeStruct((HPROWS, HID), jnp.float32),
        compiler_params=_NOTILE,
        scratch_types=[
            pltpu.VMEM((1, HD), jnp.int32),
            pltpu.VMEM((C2,), jnp.int32),
            pltpu.VMEM((C2,), jnp.int32),
            pltpu.VMEM((C2,), jnp.int32),
            pltpu.VMEM((C2,), jnp.int32),
            pltpu.VMEM((C2, HD), jnp.float32),
            pltpu.VMEM((C,), jnp.int32),
            pltpu.VMEM((C2, HID), jnp.float32),
            pltpu.VMEM((C2, HID), jnp.float32),
            pltpu.VMEM((C, HID), jnp.float32),
            pltpu.SemaphoreType.DMA,
        ],
    )
    return k(tp, g1, g2, seg, segn, keepv, hh, att)


def _pad_i32(x, length, fill):
    return jnp.concatenate(
        [x.astype(jnp.int32), jnp.full((length - x.shape[0],), fill, jnp.int32)])


def kernel(inputs, edge_index, embed_W0, embed_W1, embed_g, embed_b,
           gat_W, gat_al, gat_ar, dec_W0, dec_W1, dec_g, dec_b):
    src = edge_index[0].astype(jnp.int32)
    trg = edge_index[1].astype(jnp.int32)

    order = jnp.argsort(src).astype(jnp.int32)
    src_s = src[order]
    trg_p = trg[order]
    g1 = src[trg[order]]
    segn = jnp.concatenate([src_s[1:], jnp.full((1,), N + 1, jnp.int32)])

    a1 = _pad_i32(src_s, EU, 0)
    a2g = _pad_i32(trg_p, EU, 0)
    a2s = _pad_i32(trg_p, EU, N)
    ord_pad = _pad_i32(order, EU, E)
    g1p = _pad_i32(g1, EPAD2, 0)
    g2p = _pad_i32(trg_p, EPAD2, 0)
    segp = _pad_i32(src_s, EPAD2, N)
    segnp = _pad_i32(segn, EPAD2, N + 1)
    prev = jnp.concatenate([jnp.full((1,), -1, jnp.int32), src_s[:-1]])
    keep1 = (src_s == prev).astype(jnp.float32)
    keep16 = jnp.concatenate(
        [jnp.broadcast_to(keep1[:, None], (E, HD)),
         jnp.zeros((EPAD2 - E, HD), jnp.float32)])
    trgo = _pad_i32(trg, EU, 0)
    rmat = jnp.zeros((HD, HID), jnp.float32)
    rh = jnp.repeat(jnp.arange(HEADS), HD)
    rc = (jnp.arange(HEADS)[:, None] * HD + jnp.arange(HD)[None, :]).reshape(-1)
    rmat = rmat.at[rh, rc].set(1.0)

    targ = (jnp.arange(1, NTILES, dtype=jnp.int32) * E) // NTILES
    vals = src_s[targ]
    bmid = jnp.searchsorted(src_s, vals, side="left").astype(jnp.int32)
    B = jnp.concatenate([jnp.zeros((1,), jnp.int32), bmid,
                         jnp.full((1,), E, jnp.int32)])
    node_b = jnp.where(B[:-1] < E, src_s[jnp.minimum(B[:-1], E - 1)], N)
    node_b = node_b.at[0].set(0)
    node_hi = jnp.concatenate([node_b[1:], jnp.full((1,), N, jnp.int32)])
    tp = jnp.zeros((NTILES, HD), jnp.int32)
    tp = tp.at[:, 0].set(B[:-1]).at[:, 1].set(B[1:])
    tp = tp.at[:, 2].set(node_b).at[:, 3].set(node_hi)

    def _proj(a):
        m = jnp.zeros((HID, HD), jnp.float32)
        hs = jnp.arange(HEADS)
        rows = (hs[:, None] * HD + jnp.arange(HD)[None, :]).reshape(-1)
        cols = jnp.repeat(hs, HD)
        return m.at[rows, cols].set(a.reshape(-1))

    def _prmat(al, ar):
        return jnp.concatenate(
            [_proj(al), _proj(ar), jnp.zeros((HID, HID - 2 * HD), jnp.float32)],
            axis=1)

    g1d = embed_g.reshape(1, HID)
    b1d = embed_b.reshape(1, HID)
    gd = dec_g.reshape(1, HID)
    bd = dec_b.reshape(1, HID)

    t, st = _tc_stats(inputs, embed_W0)
    x0, hh, elr = _tc_embed(t, st, g1d, b1d, embed_W1, gat_W[0],
                            _prmat(gat_al[0], gat_ar[0]))

    hp = None
    for l in range(LAYERS):
        s, mx = _sc_p1(a1, a2g, elr)
        e, sa, sb = _sc_p2(s, a2s, ord_pad, mx)
        att16 = _sc_p2b(e, trgo, sa, sb)
        att128 = _tc_att_expand(att16[:E], rmat)
        hp = _sc_p3(tp, g1p, g2p, segp, segnp, keep16, hh, att128)
        if l + 1 < LAYERS:
            hh, elr = _tc_layer(hp[:N], x0, gat_W[l + 1],
                                _prmat(gat_al[l + 1], gat_ar[l + 1]))

    t2, st2 = _tc_dec_stats(hp[:N], x0, dec_W0)
    out = _tc_dec_out(t2, st2, gd, bd, dec_W1)
    return out


# speedup vs baseline: 1.6742x; 1.6319x over previous
import jax
import jax.numpy as jnp
from jax import lax
from jax.experimental import pallas as pl
from jax.experimental.pallas import tpu as pltpu
from jax.experimental.pallas import tpu_sc as plsc

N = 100000
E = 100000
HID = 128
HEADS = 8
HD = 16
LAYERS = 4
OUTD = 64

NTILES = 32
C = 128
C2 = 256
NCH = (E + C - 1) // C
EU = NCH * C
EPAD = E + C
EPAD2 = E + C2 + C
SROWS = 16 * 6256
HPROWS = N + C

_PREC = jax.lax.Precision.HIGHEST
_NOTILE = pltpu.CompilerParams(use_tc_tiling_on_sc=False)


def _dot(a, b):
    return jax.lax.dot_general(a, b, (((1,), (0,)), ((), ())),
                               precision=_PREC, preferred_element_type=jnp.float32)


_RB = 1000
_NB = N // _RB


def _tc_stats_body(x_ref, w_ref, t_ref, st_ref, acc_ref):
    i = pl.program_id(0)
    t = _dot(x_ref[...], w_ref[...])
    t_ref[...] = t
    s0 = jnp.sum(t, axis=0, keepdims=True)
    s1 = jnp.sum(t * t, axis=0, keepdims=True)
    blk = jnp.concatenate([s0, s1], axis=0)

    @pl.when(i == 0)
    def _():
        acc_ref[...] = jnp.zeros_like(acc_ref)

    acc_ref[...] += blk
    st_ref[...] = acc_ref[...]


def _tc_stats(x, w):
    return pl.pallas_call(
        _tc_stats_body,
        grid=(_NB,),
        in_specs=[
            pl.BlockSpec((_RB, HID), lambda i: (i, 0)),
            pl.BlockSpec((HID, HID), lambda i: (0, 0)),
        ],
        out_specs=[
            pl.BlockSpec((_RB, HID), lambda i: (i, 0)),
            pl.BlockSpec((2, HID), lambda i: (0, 0)),
        ],
        out_shape=[
            jax.ShapeDtypeStruct((N, HID), jnp.float32),
            jax.ShapeDtypeStruct((2, HID), jnp.float32),
        ],
        scratch_shapes=[pltpu.VMEM((2, HID), jnp.float32)],
    )(x, w)


def _bn_act(t, st, g, b):
    mu = st[0:1, :] / N
    var = st[1:2, :] / N - mu * mu
    xn = (t - mu) * jax.lax.rsqrt(var + 1e-5)
    return jnp.maximum(g * xn + b, 0.0)


def _tc_embed_body(t_ref, st_ref, g_ref, b_ref, w1_ref, wg_ref, pr_ref,
                   x0_ref, hh_ref, elr_ref):
    a = _bn_act(t_ref[...], st_ref[...], g_ref[...], b_ref[...])
    x0 = _dot(a, w1_ref[...])
    x0_ref[...] = x0
    hh = _dot(x0, wg_ref[...])
    hh_ref[...] = hh
    elr_ref[...] = _dot(hh, pr_ref[...])


def _tc_embed(t, st, g, b, w1, wg, pr):
    return pl.pallas_call(
        _tc_embed_body,
        grid=(_NB,),
        in_specs=[
            pl.BlockSpec((_RB, HID), lambda i: (i, 0)),
            pl.BlockSpec((2, HID), lambda i: (0, 0)),
            pl.BlockSpec((1, HID), lambda i: (0, 0)),
            pl.BlockSpec((1, HID), lambda i: (0, 0)),
            pl.BlockSpec((HID, HID), lambda i: (0, 0)),
            pl.BlockSpec((HID, HID), lambda i: (0, 0)),
            pl.BlockSpec((HID, HID), lambda i: (0, 0)),
        ],
        out_specs=[
            pl.BlockSpec((_RB, HID), lambda i: (i, 0)),
            pl.BlockSpec((_RB, HID), lambda i: (i, 0)),
            pl.BlockSpec((_RB, HID), lambda i: (i, 0)),
        ],
        out_shape=[
            jax.ShapeDtypeStruct((N, HID), jnp.float32),
            jax.ShapeDtypeStruct((N, HID), jnp.float32),
            jax.ShapeDtypeStruct((N, HID), jnp.float32),
        ],
    )(t, st, g, b, w1, wg, pr)


def _tc_layer_body(hp_ref, x0_ref, wg_ref, pr_ref, hh_ref, elr_ref):
    hp = hp_ref[...]
    h = jnp.maximum(hp, 0.0) + 0.01 * jnp.minimum(hp, 0.0) + x0_ref[...]
    hh = _dot(h, wg_ref[...])
    hh_ref[...] = hh
    elr_ref[...] = _dot(hh, pr_ref[...])


def _tc_layer(hp, x0, wg, pr):
    return pl.pallas_call(
        _tc_layer_body,
        grid=(_NB,),
        in_specs=[
            pl.BlockSpec((_RB, HID), lambda i: (i, 0)),
            pl.BlockSpec((_RB, HID), lambda i: (i, 0)),
            pl.BlockSpec((HID, HID), lambda i: (0, 0)),
            pl.BlockSpec((HID, HID), lambda i: (0, 0)),
        ],
        out_specs=[
            pl.BlockSpec((_RB, HID), lambda i: (i, 0)),
            pl.BlockSpec((_RB, HID), lambda i: (i, 0)),
        ],
        out_shape=[
            jax.ShapeDtypeStruct((N, HID), jnp.float32),
            jax.ShapeDtypeStruct((N, HID), jnp.float32),
        ],
    )(hp, x0, wg, pr)


def _tc_dec_stats_body(hp_ref, x0_ref, w_ref, t_ref, st_ref, acc_ref):
    i = pl.program_id(0)
    hp = hp_ref[...]
    h = jnp.maximum(hp, 0.0) + 0.01 * jnp.minimum(hp, 0.0) + x0_ref[...]
    t = _dot(h, w_ref[...])
    t_ref[...] = t
    s0 = jnp.sum(t, axis=0, keepdims=True)
    s1 = jnp.sum(t * t, axis=0, keepdims=True)
    blk = jnp.concatenate([s0, s1], axis=0)

    @pl.when(i == 0)
    def _():
        acc_ref[...] = jnp.zeros_like(acc_ref)

    acc_ref[...] += blk
    st_ref[...] = acc_ref[...]


def _tc_dec_stats(hp, x0, w):
    return pl.pallas_call(
        _tc_dec_stats_body,
        grid=(_NB,),
        in_specs=[
            pl.BlockSpec((_RB, HID), lambda i: (i, 0)),
            pl.BlockSpec((_RB, HID), lambda i: (i, 0)),
            pl.BlockSpec((HID, HID), lambda i: (0, 0)),
        ],
        out_specs=[
            pl.BlockSpec((_RB, HID), lambda i: (i, 0)),
            pl.BlockSpec((2, HID), lambda i: (0, 0)),
        ],
        out_shape=[
            jax.ShapeDtypeStruct((N, HID), jnp.float32),
            jax.ShapeDtypeStruct((2, HID), jnp.float32),
        ],
        scratch_shapes=[pltpu.VMEM((2, HID), jnp.float32)],
    )(hp, x0, w)


def _tc_dec_out_body(t_ref, st_ref, g_ref, b_ref, w1_ref, o_ref):
    a = _bn_act(t_ref[...], st_ref[...], g_ref[...], b_ref[...])
    o_ref[...] = _dot(a, w1_ref[...])


def _tc_dec_out(t, st, g, b, w1):
    return pl.pallas_call(
        _tc_dec_out_body,
        grid=(_NB,),
        in_specs=[
            pl.BlockSpec((_RB, HID), lambda i: (i, 0)),
            pl.BlockSpec((2, HID), lambda i: (0, 0)),
            pl.BlockSpec((1, HID), lambda i: (0, 0)),
            pl.BlockSpec((1, HID), lambda i: (0, 0)),
            pl.BlockSpec((HID, OUTD), lambda i: (0, 0)),
        ],
        out_specs=pl.BlockSpec((_RB, OUTD), lambda i: (i, 0)),
        out_shape=jax.ShapeDtypeStruct((N, OUTD), jnp.float32),
    )(t, st, g, b, w1)


_MESH = plsc.VectorSubcoreMesh(core_axis_name="c", subcore_axis_name="s")


def _lane():
    return lax.iota(jnp.int32, 16)


def _sc_p1_body(a1_hbm, a2_hbm, elr_hbm, s_hbm, mx_hbm,
                i1_v, i2_v, r1_v, r2_v, s_v, m_v):
    w = lax.axis_index("s") * 2 + lax.axis_index("c")
    nck = (NCH + 31 - w) // 32
    neg = jnp.full((16,), -3.0e38, jnp.float32)
    head = _lane() < jnp.full((16,), HEADS, jnp.int32)

    def chunk(k, macc):
        base = (k * 32 + w) * C
        nv = jnp.minimum(C, E - base)
        pltpu.sync_copy(a1_hbm.at[pl.ds(base, C)], i1_v)
        pltpu.sync_copy(a2_hbm.at[pl.ds(base, C)], i2_v)
        pltpu.sync_copy(elr_hbm.at[i1_v], r1_v)
        pltpu.sync_copy(elr_hbm.at[i2_v], r2_v)

        def row(r, acc):
            x = r1_v[r, 0:16] + r2_v[r, 16:32]
            s = jnp.maximum(x, 0.0) + 0.2 * jnp.minimum(x, 0.0)
            s_v[r] = s
            return jnp.maximum(acc, jnp.where(head, s, neg))

        macc = lax.fori_loop(0, nv, row, macc)
        pltpu.sync_copy(s_v, s_hbm.at[pl.ds(base, C)])
        return macc

    macc = lax.fori_loop(0, nck, chunk, neg)
    m_v[0] = macc
    pltpu.sync_copy(m_v, mx_hbm.at[pl.ds(w, 1)])


def _sc_p1(a1, a2, elr):
    k = pl.kernel(
        _sc_p1_body,
        mesh=_MESH,
        out_type=[
            jax.ShapeDtypeStruct((EU, HD), jnp.float32),
            jax.ShapeDtypeStruct((NTILES, HD), jnp.float32),
        ],
        compiler_params=_NOTILE,
        scratch_types=[
            pltpu.VMEM((C,), jnp.int32),
            pltpu.VMEM((C,), jnp.int32),
            pltpu.VMEM((C, HID), jnp.float32),
            pltpu.VMEM((C, HID), jnp.float32),
            pltpu.VMEM((C, HD), jnp.float32),
            pltpu.VMEM((1, HD), jnp.float32),
        ],
    )
    return k(a1, a2, elr)


def _sc_p2_body(s_hbm, trg_hbm, ord_hbm, mx_hbm, e_hbm, sa_hbm, sb_hbm,
                sums_sh, s_v, e_v, trg_v, ord_v, mx_v, z_v):
    w = lax.axis_index("s") * 2 + lax.axis_index("c")
    core = lax.axis_index("c")
    sub = lax.axis_index("s")
    rows_per = SROWS // 16

    pltpu.sync_copy(mx_hbm, mx_v)
    macc = mx_v[0]
    for i in range(1, NTILES):
        macc = jnp.maximum(macc, mx_v[i])
    m = macc[0]
    for i in range(1, 16):
        m = jnp.maximum(m, macc[i])

    def zrow(r, _):
        z_v[r] = jnp.zeros((16,), jnp.float32)
        return 0

    lax.fori_loop(0, C, zrow, 0)
    nzc = rows_per // C
    rem = rows_per - nzc * C

    def zchunk(j, _):
        pltpu.sync_copy(z_v, sums_sh.at[pl.ds(sub * rows_per + j * C, C)])
        return 0

    lax.fori_loop(0, nzc, zchunk, 0)
    if rem:
        pltpu.sync_copy(z_v.at[pl.ds(0, rem)],
                        sums_sh.at[pl.ds(sub * rows_per + nzc * C, rem)])
    plsc.subcore_barrier()

    nck = (NCH + 31 - w) // 32

    def chunk(k, _):
        base = (k * 32 + w) * C
        pltpu.sync_copy(s_hbm.at[pl.ds(base, C)], s_v)
        pltpu.sync_copy(trg_hbm.at[pl.ds(base, C)], trg_v)
        pltpu.sync_copy(ord_hbm.at[pl.ds(base, C)], ord_v)

        def row(r, _):
            e_v[r] = jnp.exp(s_v[r] - m)
            return 0

        lax.fori_loop(0, C, row, 0)
        pltpu.sync_copy(e_v, sums_sh.at[trg_v], add=True)
        pltpu.sync_copy(e_v, e_hbm.at[ord_v])
        return 0

    lax.fori_loop(0, nck, chunk, 0)
    plsc.subcore_barrier()

    @pl.when(core == 0)
    def _():
        pltpu.sync_copy(sums_sh.at[pl.ds(sub * rows_per, rows_per)],
                        sa_hbm.at[pl.ds(sub * rows_per, rows_per)])

    @pl.when(core == 1)
    def _():
        pltpu.sync_copy(sums_sh.at[pl.ds(sub * rows_per, rows_per)],
                        sb_hbm.at[pl.ds(sub * rows_per, rows_per)])


def _sc_p2(s, trg_s, ord_s, mx):
    k = pl.kernel(
        _sc_p2_body,
        mesh=_MESH,
        out_type=[
            jax.ShapeDtypeStruct((EPAD, HD), jnp.float32),
            jax.ShapeDtypeStruct((SROWS, HD), jnp.float32),
            jax.ShapeDtypeStruct((SROWS, HD), jnp.float32),
        ],
        compiler_params=_NOTILE,
        scratch_types=[
            pltpu.VMEM_SHARED((SROWS, HD), jnp.float32),
            pltpu.VMEM((C, HD), jnp.float32),
            pltpu.VMEM((C, HD), jnp.float32),
            pltpu.VMEM((C,), jnp.int32),
            pltpu.VMEM((C,), jnp.int32),
            pltpu.VMEM((NTILES, HD), jnp.float32),
            pltpu.VMEM((C, HD), jnp.float32),
        ],
    )
    return k(s, trg_s, ord_s, mx)


def _sc_p2b_body(e_hbm, trg_hbm, sa_hbm, sb_hbm, att_hbm,
                 e_v, trg_v, sa_v, sb_v, att_v, sem):
    w = lax.axis_index("s") * 2 + lax.axis_index("c")
    nck = (NCH + 31 - w) // 32

    def chunk(k, _):
        base = (k * 32 + w) * C
        cps = [
            pltpu.async_copy(e_hbm.at[pl.ds(base, C)], e_v, sem),
            pltpu.async_copy(trg_hbm.at[pl.ds(base, C)], trg_v, sem),
        ]
        for cp in cps:
            cp.wait()
        gs = [
            pltpu.async_copy(sa_hbm.at[trg_v], sa_v, sem),
            pltpu.async_copy(sb_hbm.at[trg_v], sb_v, sem),
        ]
        for cp in gs:
            cp.wait()

        def row(r, _):
            att_v[r] = e_v[r] / (sa_v[r] + sb_v[r] + 1e-16)
            return 0

        lax.fori_loop(0, C, row, 0)
        pltpu.sync_copy(att_v, att_hbm.at[pl.ds(base, C)])
        return 0

    lax.fori_loop(0, nck, chunk, 0)


def _sc_p2b(e, trg_o, sa, sb):
    k = pl.kernel(
        _sc_p2b_body,
        mesh=_MESH,
        out_type=jax.ShapeDtypeStruct((EU, HD), jnp.float32),
        compiler_params=_NOTILE,
        scratch_types=[
            pltpu.VMEM((C, HD), jnp.float32),
            pltpu.VMEM((C,), jnp.int32),
            pltpu.VMEM((C, HD), jnp.float32),
            pltpu.VMEM((C, HD), jnp.float32),
            pltpu.VMEM((C, HD), jnp.float32),
            pltpu.SemaphoreType.DMA,
        ],
    )
    return k(e, trg_o, sa, sb)


def _tc_att_expand_body(a_ref, r_ref, o_ref):
    o_ref[...] = _dot(a_ref[...], r_ref[...])


def _tc_att_expand(att16, rmat):
    return pl.pallas_call(
        _tc_att_expand_body,
        grid=(_NB,),
        in_specs=[
            pl.BlockSpec((_RB, HD), lambda i: (i, 0)),
            pl.BlockSpec((HD, HID), lambda i: (0, 0)),
        ],
        out_specs=pl.BlockSpec((_RB, HID), lambda i: (i, 0)),
        out_shape=jax.ShapeDtypeStruct((E, HID), jnp.float32),
    )(att16, rmat)


W = 512


def _sc_p3_body(tp_hbm, g1_hbm, g2_hbm, seg_hbm, keep_hbm,
                hh_hbm, att_hbm, hp_hbm,
                tp_v, g1_v, g2_v, seg_v, keep_v, ids_v,
                hh_v, att_v, stg_v, sem):
    w = lax.axis_index("s") * 2 + lax.axis_index("c")
    lane = _lane()
    nfull = jnp.full((16,), N, jnp.int32)

    pltpu.sync_copy(tp_hbm.at[pl.ds(w, 1)], tp_v)
    trow = tp_v[0]
    b0 = trow[0]
    b1 = trow[1]
    nb0 = trow[2]
    nb1 = trow[3]

    def zrow(r, _):
        for h in range(HEADS):
            stg_v[r, 16 * h:16 * (h + 1)] = jnp.zeros((16,), jnp.float32)
        return 0

    def rezero():
        lax.fori_loop(0, W, zrow, 0)

    rezero()

    ab0 = (b0 // 8) * 8
    ncks = (b1 - ab0 + C - 1) // C

    def wflush(_, wbx):
        pltpu.sync_copy(stg_v, hp_hbm.at[pl.ds(wbx, W)])
        rezero()
        return wbx + W

    def chunk(ck, carry):
        accs, wb = carry
        ab = ab0 + ck * C
        nv = jnp.clip(b1 - ab, 0, C)
        lo = jnp.clip(b0 - ab, 0, C)
        cps = [
            pltpu.async_copy(g1_hbm.at[pl.ds(ab, C)], g1_v, sem),
            pltpu.async_copy(g2_hbm.at[pl.ds(ab, C)], g2_v, sem),
            pltpu.async_copy(seg_hbm.at[pl.ds(ab, C)], seg_v.at[pl.ds(0, C)], sem),
            pltpu.async_copy(keep_hbm.at[pl.ds(ab, C)], keep_v, sem),
        ]
        for cp in cps:
            cp.wait()
        gs = [
            pltpu.async_copy(hh_hbm.at[g1_v], hh_v, sem),
            pltpu.async_copy(att_hbm.at[g2_v], att_v, sem),
        ]
        for cp in gs:
            cp.wait()

        def edge(r, ecarry):
            eaccs, ewb = ecarry
            seg = seg_v[pl.ds(r, 16)][0]
            nadv = (seg - ewb) // W
            ewb = lax.fori_loop(0, nadv, wflush, ewb)
            slot = seg - ewb
            kv = keep_v[r]
            new = []
            for h in range(HEADS):
                hv = hh_v[r, 16 * h:16 * (h + 1)]
                av = att_v[r, 16 * h:16 * (h + 1)]
                ya = hv * av + kv * eaccs[h]
                stg_v[slot, 16 * h:16 * (h + 1)] = ya
                new.append(ya)
            return tuple(new), ewb

        return lax.fori_loop(lo, nv, edge, (accs, wb))

    init = (tuple(jnp.zeros((16,), jnp.float32) for _ in range(HEADS)), nb0)
    _, wb = lax.fori_loop(0, ncks, chunk, init)

    wb = lax.fori_loop(0, (nb1 - wb) // W, wflush, wb)
    for t in range(W // C):
        for j in range(C // 16):
            pos = wb + jnp.full((16,), t * C + 16 * j, jnp.int32) + lane
            ids_v[16 * j:16 * j + 16] = jnp.where(
                pos < jnp.full((16,), 1, jnp.int32) * nb1, pos, nfull)
        pltpu.sync_copy(stg_v.at[pl.ds(t * C, C)], hp_hbm.at[ids_v])


def _sc_p3(tp, g1, g2, seg, keepv, hh, att):
    k = pl.kernel(
        _sc_p3_body,
        mesh=_MESH,
        out_type=jax.ShapeDtypeStruct((HPROWS, HID), jnp.float32),
        compiler_params=_NOTILE,
        scratch_types=[
            pltpu.VMEM((1, HD), jnp.int32),
            pltpu.VMEM((C,), jnp.int32),
            pltpu.VMEM((C,), jnp.int32),
            pltpu.VMEM((C + 16,), jnp.int32),
            pltpu.VMEM((C, HD), jnp.float32),
            pltpu.VMEM((C,), jnp.int32),
            pltpu.VMEM((C, HID), jnp.float32),
            pltpu.VMEM((C, HID), jnp.float32),
            pltpu.VMEM((W, HID), jnp.float32),
            pltpu.SemaphoreType.DMA,
        ],
    )
    return k(tp, g1, g2, seg, keepv, hh, att)


def _pad_i32(x, length, fill):
    return jnp.concatenate(
        [x.astype(jnp.int32), jnp.full((length - x.shape[0],), fill, jnp.int32)])


def kernel(inputs, edge_index, embed_W0, embed_W1, embed_g, embed_b,
           gat_W, gat_al, gat_ar, dec_W0, dec_W1, dec_g, dec_b):
    src = edge_index[0].astype(jnp.int32)
    trg = edge_index[1].astype(jnp.int32)

    order = jnp.argsort(src).astype(jnp.int32)
    src_s = src[order]
    trg_p = trg[order]
    g1 = src[trg[order]]

    a1 = _pad_i32(src_s, EU, 0)
    a2g = _pad_i32(trg_p, EU, 0)
    a2s = _pad_i32(trg_p, EU, N)
    ord_pad = _pad_i32(order, EU, E)
    g1p = _pad_i32(g1, EPAD2, 0)
    g2p = _pad_i32(trg_p, EPAD2, 0)
    segp = _pad_i32(src_s, EPAD2, N)
    prev = jnp.concatenate([jnp.full((1,), -1, jnp.int32), src_s[:-1]])
    keep1 = (src_s == prev).astype(jnp.float32)
    keep16 = jnp.concatenate(
        [jnp.broadcast_to(keep1[:, None], (E, HD)),
         jnp.zeros((EPAD2 - E, HD), jnp.float32)])
    trgo = _pad_i32(trg, EU, 0)
    rmat = jnp.zeros((HD, HID), jnp.float32)
    rh = jnp.repeat(jnp.arange(HEADS), HD)
    rc = (jnp.arange(HEADS)[:, None] * HD + jnp.arange(HD)[None, :]).reshape(-1)
    rmat = rmat.at[rh, rc].set(1.0)

    targ = (jnp.arange(1, NTILES, dtype=jnp.int32) * E) // NTILES
    vals = src_s[targ]
    bmid = jnp.searchsorted(src_s, vals, side="left").astype(jnp.int32)
    B = jnp.concatenate([jnp.zeros((1,), jnp.int32), bmid,
                         jnp.full((1,), E, jnp.int32)])
    node_b = jnp.where(B[:-1] < E, src_s[jnp.minimum(B[:-1], E - 1)], N)
    node_b = node_b.at[0].set(0)
    node_hi = jnp.concatenate([node_b[1:], jnp.full((1,), N, jnp.int32)])
    tp = jnp.zeros((NTILES, HD), jnp.int32)
    tp = tp.at[:, 0].set(B[:-1]).at[:, 1].set(B[1:])
    tp = tp.at[:, 2].set(node_b).at[:, 3].set(node_hi)

    def _proj(a):
        m = jnp.zeros((HID, HD), jnp.float32)
        hs = jnp.arange(HEADS)
        rows = (hs[:, None] * HD + jnp.arange(HD)[None, :]).reshape(-1)
        cols = jnp.repeat(hs, HD)
        return m.at[rows, cols].set(a.reshape(-1))

    def _prmat(al, ar):
        return jnp.concatenate(
            [_proj(al), _proj(ar), jnp.zeros((HID, HID - 2 * HD), jnp.float32)],
            axis=1)

    g1d = embed_g.reshape(1, HID)
    b1d = embed_b.reshape(1, HID)
    gd = dec_g.reshape(1, HID)
    bd = dec_b.reshape(1, HID)

    t, st = _tc_stats(inputs, embed_W0)
    x0, hh, elr = _tc_embed(t, st, g1d, b1d, embed_W1, gat_W[0],
                            _prmat(gat_al[0], gat_ar[0]))

    hp = None
    for l in range(LAYERS):
        s, mx = _sc_p1(a1, a2g, elr)
        e, sa, sb = _sc_p2(s, a2s, ord_pad, mx)
        att16 = _sc_p2b(e, trgo, sa, sb)
        att128 = _tc_att_expand(att16[:E], rmat)
        hp = _sc_p3(tp, g1p, g2p, segp, keep16, hh, att128)
        if l + 1 < LAYERS:
            hh, elr = _tc_layer(hp[:N], x0, gat_W[l + 1],
                                _prmat(gat_al[l + 1], gat_ar[l + 1]))

    t2, st2 = _tc_dec_stats(hp[:N], x0, dec_W0)
    out = _tc_dec_out(t2, st2, gd, bd, dec_W1)
    return out
